# baseline (speedup 1.0000x reference)
import functools

import jax
import jax.numpy as jnp
import numpy as np
from jax import lax
from jax.experimental import pallas as pl
from jax.experimental.pallas import tpu as pltpu
from jax.experimental.pallas import tpu_sc as plsc

BATCH = 16384
D_NUM = 13
N_CAT = 26
VOCAB = 100000
D_MODEL = 16
N_TOK = D_NUM + N_CAT

CHUNK_B = 64
CHUNK_IDX = CHUNK_B * N_CAT
IDX_ROWS = CHUNK_IDX // 128
CHUNK_OUT = CHUNK_B * N_TOK * D_MODEL

TBL_ROWS = N_CAT * VOCAB
BLK_ROWS = TBL_ROWS // 8
RELAYOUT_BR = 8000


def _pe_const():
    pos = np.arange(N_TOK, dtype=np.float32)[:, None]
    i2 = np.arange(0, D_MODEL, 2, dtype=np.float32)
    pe = np.zeros((N_TOK, D_MODEL), dtype=np.float32)
    pe[:, ::2] = np.sin(pos / 10000.0 ** (i2 / D_MODEL))
    pe[:, 1::2] = np.cos(pos / 10000.0 ** (i2 / D_MODEL))
    return pe


def _relayout_table(tbl_flat):
    def body(x_ref, o_ref):
        o_ref[...] = x_ref[...].reshape(RELAYOUT_BR * D_MODEL // 128, 128)

    return pl.pallas_call(
        body,
        grid=(TBL_ROWS // RELAYOUT_BR,),
        in_specs=[pl.BlockSpec((RELAYOUT_BR * D_MODEL,), lambda i: (i,))],
        out_specs=pl.BlockSpec((RELAYOUT_BR * D_MODEL // 128, 128),
                               lambda i: (i, 0)),
        out_shape=jax.ShapeDtypeStruct((BLK_ROWS, 128), jnp.float32),
    )(tbl_flat)


def kernel(x_num, x_cat, weight, bias, emb_table):
    info = plsc.get_sparse_core_info()
    nc, ns = info.num_cores, info.num_subcores
    nw = nc * ns
    b_per_w = BATCH // nw
    n_chunks = b_per_w // CHUNK_B
    idx_rows_w = b_per_w * N_CAT // 128

    xcat2d = (
        x_cat.astype(jnp.int32)
        .reshape(nw, n_chunks, CHUNK_B, N_CAT)
        .transpose(0, 1, 3, 2)
        .reshape(BATCH * N_CAT // 128, 128)
    )
    xnum_flat = jnp.pad(x_num, ((0, 0), (0, 16 - D_NUM))).reshape(-1)
    pe = jnp.asarray(_pe_const())
    off2d = jnp.asarray(
        ((np.arange(CHUNK_IDX, dtype=np.int32) // CHUNK_B) * VOCAB)
        .reshape(IDX_ROWS, 128)
    )

    tblr = _relayout_table(emb_table.reshape(-1))

    mesh = plsc.VectorSubcoreMesh(core_axis_name="c", subcore_axis_name="s")

    @functools.partial(
        pl.kernel,
        out_type=jax.ShapeDtypeStruct((BATCH * N_TOK * D_MODEL,),
                                      jnp.float32),
        mesh=mesh,
        scratch_types=[
            pltpu.VMEM((104, 128), jnp.int32),
            pltpu.VMEM((104, 128), jnp.int32),
            pltpu.VMEM((IDX_ROWS, 128), jnp.int32),
            pltpu.VMEM((2, 128, 128), jnp.float32),
            pltpu.VMEM((CHUNK_OUT,), jnp.float32),
            pltpu.VMEM((CHUNK_B * 16,), jnp.float32),
            pltpu.VMEM((D_NUM, D_MODEL), jnp.float32),
            pltpu.VMEM((N_TOK, D_MODEL), jnp.float32),
            pltpu.VMEM((N_TOK, D_MODEL), jnp.float32),
            pltpu.SemaphoreType.DMA,
        ],
    )
    def sc_embed(xcat_hbm, xnum_hbm, w_hbm, bias_hbm, pe_hbm, off_hbm,
                 table_hbm, out_hbm,
                 idx_v, sub_v, off_v, blocks_v, out_v, xnum_v, w_v, av_v,
                 pe_v, sem):
        wid = lax.axis_index("s") * nc + lax.axis_index("c")

        pltpu.sync_copy(w_hbm, w_v)
        pltpu.sync_copy(bias_hbm, av_v)
        pltpu.sync_copy(pe_hbm, pe_v)
        pltpu.sync_copy(off_hbm, off_v)
        for i in range(N_TOK):
            av_v[i, :] = av_v[i, :] + pe_v[i, :]

        pltpu.sync_copy(xcat_hbm.at[pl.ds(wid * idx_rows_w, idx_rows_w)], idx_v)

        def off_body(g, carry):
            for r in range(IDX_ROWS):
                for k in range(128 // 16):
                    sl = pl.ds(k * 16, 16)
                    v = idx_v[g * IDX_ROWS + r, sl] + off_v[r, sl]
                    sub_v[g * IDX_ROWS + r, sl] = (v & 7) * D_MODEL
                    idx_v[g * IDX_ROWS + r, sl] = v >> 3
            return carry

        lax.fori_loop(0, n_chunks, off_body, 0)

        def gather_row(row, buf):
            return pltpu.async_copy(
                table_hbm.at[idx_v.at[row]], blocks_v.at[buf], sem
            )

        def chunk_body(t, carry):
            b0 = wid * b_per_w + t * CHUNK_B
            row0 = t * IDX_ROWS

            pltpu.sync_copy(xnum_hbm.at[pl.ds(b0 * 16, CHUNK_B * 16)], xnum_v)

            cp = gather_row(row0, 0)

            def num_body(b, c2):
                xv = xnum_v[pl.ds(b * 16, 16)]
                q = b * N_TOK
                for j in range(D_NUM):
                    out_v[pl.ds((q + j) * D_MODEL, D_MODEL)] = (
                        xv[j] * w_v[j, :] + av_v[j, :]
                    )
                return c2

            lax.fori_loop(0, CHUNK_B, num_body, 0)

            for r in range(IDX_ROWS):
                if r + 1 < IDX_ROWS:
                    cp_next = gather_row(row0 + r + 1, (r + 1) % 2)
                cp.wait()
                buf = r % 2
                for half in range(2):
                    c = 2 * r + half
                    avc = av_v[D_NUM + c, :]

                    def ex_body(g, c2, _half=half, _buf=buf, _c=c, _avc=avc):
                        subs = sub_v[row0 + r, pl.ds(_half * 64 + g * 16, 16)]
                        for k in range(16):
                            i = _half * 64 + g * 16 + k
                            t_ = (g * 16 + k) * N_TOK + D_NUM + _c
                            out_v[pl.ds(t_ * D_MODEL, D_MODEL)] = (
                                blocks_v[_buf, i, pl.ds(subs[k], D_MODEL)]
                                + _avc
                            )
                        return c2

                    lax.fori_loop(0, 4, ex_body, 0)
                if r + 1 < IDX_ROWS:
                    cp = cp_next

            pltpu.sync_copy(
                out_v, out_hbm.at[pl.ds((wid * n_chunks + t) * CHUNK_OUT,
                                        CHUNK_OUT)]
            )
            return carry

        lax.fori_loop(0, n_chunks, chunk_body, 0)

    flat = sc_embed(xcat2d, xnum_flat, weight, bias, pe, off2d, tblr)
    return flat.reshape(BATCH, N_TOK, D_MODEL)


# speedup vs baseline: 1.1931x; 1.1931x over previous
import functools

import jax
import jax.numpy as jnp
import numpy as np
from jax import lax
from jax.experimental import pallas as pl
from jax.experimental.pallas import tpu as pltpu
from jax.experimental.pallas import tpu_sc as plsc

BATCH = 16384
D_NUM = 13
N_CAT = 26
VOCAB = 100000
D_MODEL = 16
N_TOK = D_NUM + N_CAT

CHUNK_B = 64
CHUNK_IDX = CHUNK_B * N_CAT
IDX_ROWS = CHUNK_IDX // 128
CHUNK_OUT = CHUNK_B * N_TOK * D_MODEL


def _pe_const():
    pos = np.arange(N_TOK, dtype=np.float32)[:, None]
    i2 = np.arange(0, D_MODEL, 2, dtype=np.float32)
    pe = np.zeros((N_TOK, D_MODEL), dtype=np.float32)
    pe[:, ::2] = np.sin(pos / 10000.0 ** (i2 / D_MODEL))
    pe[:, 1::2] = np.cos(pos / 10000.0 ** (i2 / D_MODEL))
    return pe


def kernel(x_num, x_cat, weight, bias, emb_table):
    info = plsc.get_sparse_core_info()
    nc, ns = info.num_cores, info.num_subcores
    nw = nc * ns
    b_per_w = BATCH // nw
    n_chunks = b_per_w // CHUNK_B
    idx_rows_w = b_per_w * N_CAT // 128

    xcat2d = x_cat.astype(jnp.int32).reshape(BATCH * N_CAT // 128, 128)
    xnum_flat = jnp.pad(x_num, ((0, 0), (0, 16 - D_NUM))).reshape(-1)
    pe = jnp.asarray(_pe_const())
    off2d = jnp.asarray(
        (np.arange(CHUNK_IDX, dtype=np.int32) % N_CAT) * VOCAB
    ).reshape(IDX_ROWS, 128)

    mesh = plsc.VectorSubcoreMesh(core_axis_name="c", subcore_axis_name="s")

    @functools.partial(
        pl.kernel,
        out_type=jax.ShapeDtypeStruct((BATCH * N_TOK * D_MODEL,),
                                      jnp.float32),
        mesh=mesh,
        scratch_types=[
            pltpu.VMEM((104, 128), jnp.int32),
            pltpu.VMEM((IDX_ROWS, 128), jnp.int32),
            pltpu.VMEM((CHUNK_IDX, D_MODEL), jnp.float32),
            pltpu.VMEM((CHUNK_OUT,), jnp.float32),
            pltpu.VMEM((CHUNK_B * 16,), jnp.float32),
            pltpu.VMEM((D_NUM, D_MODEL), jnp.float32),
            pltpu.VMEM((N_TOK, D_MODEL), jnp.float32),
            pltpu.VMEM((N_TOK, D_MODEL), jnp.float32),
            pltpu.SemaphoreType.DMA,
        ],
        compiler_params=pltpu.CompilerParams(use_tc_tiling_on_sc=False),
    )
    def sc_embed(xcat_hbm, xnum_hbm, w_hbm, bias_hbm, pe_hbm, off_hbm,
                 table_hbm, out_hbm,
                 idx_v, off_v, rows_v, out_v, xnum_v, w_v, av_v, pe_v, sem):
        wid = lax.axis_index("s") * nc + lax.axis_index("c")

        pltpu.sync_copy(w_hbm, w_v)
        pltpu.sync_copy(bias_hbm, av_v)
        pltpu.sync_copy(pe_hbm, pe_v)
        pltpu.sync_copy(off_hbm, off_v)
        for i in range(N_TOK):
            av_v[i, :] = av_v[i, :] + pe_v[i, :]

        pltpu.sync_copy(xcat_hbm.at[pl.ds(wid * idx_rows_w, idx_rows_w)], idx_v)

        def off_body(g, carry):
            for r in range(IDX_ROWS):
                for k in range(128 // 16):
                    sl = pl.ds(k * 16, 16)
                    idx_v[g * IDX_ROWS + r, sl] = (
                        idx_v[g * IDX_ROWS + r, sl] + off_v[r, sl]
                    )
            return carry

        lax.fori_loop(0, n_chunks, off_body, 0)

        def chunk_body(t, carry):
            b0 = wid * b_per_w + t * CHUNK_B

            pltpu.sync_copy(xnum_hbm.at[pl.ds(b0 * 16, CHUNK_B * 16)], xnum_v)

            cps = []
            for r in range(IDX_ROWS):
                cps.append(
                    pltpu.async_copy(
                        table_hbm.at[idx_v.at[t * IDX_ROWS + r]],
                        rows_v.at[pl.ds(r * 128, 128)],
                        sem,
                    )
                )
            for cp in cps:
                cp.wait()

            def b_body(b, c2):
                xv = xnum_v[pl.ds(b * 16, 16)]
                q = b * N_TOK
                for j in range(D_NUM):
                    out_v[pl.ds((q + j) * D_MODEL, D_MODEL)] = (
                        xv[j] * w_v[j, :] + av_v[j, :]
                    )
                for c in range(N_CAT):
                    out_v[pl.ds((q + D_NUM + c) * D_MODEL, D_MODEL)] = (
                        rows_v[b * N_CAT + c, :] + av_v[D_NUM + c, :]
                    )
                return c2

            lax.fori_loop(0, CHUNK_B, b_body, 0)

            pltpu.sync_copy(
                out_v, out_hbm.at[pl.ds((wid * n_chunks + t) * CHUNK_OUT,
                                        CHUNK_OUT)]
            )
            return carry

        lax.fori_loop(0, n_chunks, chunk_body, 0)

    flat = sc_embed(xcat2d, xnum_flat, weight, bias, pe, off2d, emb_table)
    return flat.reshape(BATCH, N_TOK, D_MODEL)
